# baseline (device time: 24403 ns/iter reference)
import functools

import jax
import jax.numpy as jnp
from jax import lax
from jax.experimental import pallas as pl
from jax.experimental.pallas import tpu as pltpu

N_DEV = 4
B, SQ_PER, SKV_PER, HQ, DH = 2, 128, 128, 4, 64
D_MODEL = 512
D_QK = HQ * DH
BLK = 64


def kernel(x, Wq, K_ext, V_ext, Wo):
    def body(x_ref, wq_ref, k_ref, v_ref, wo_ref, out_ref,
             ck_ref, cv_ref, send_sems, recv_sems):
        my = lax.axis_index("i")
        partner = (my + 2) % N_DEV

        barrier_sem = pltpu.get_barrier_semaphore()
        pl.semaphore_signal(
            barrier_sem, inc=1,
            device_id=(partner,), device_id_type=pl.DeviceIdType.MESH,
        )
        pl.semaphore_wait(barrier_sem, 1)

        rdma_k = pltpu.make_async_remote_copy(
            src_ref=k_ref, dst_ref=ck_ref,
            send_sem=send_sems.at[0], recv_sem=recv_sems.at[0],
            device_id=(partner,), device_id_type=pl.DeviceIdType.MESH,
        )
        rdma_v = pltpu.make_async_remote_copy(
            src_ref=v_ref, dst_ref=cv_ref,
            send_sem=send_sems.at[1], recv_sem=recv_sems.at[1],
            device_id=(partner,), device_id_type=pl.DeviceIdType.MESH,
        )
        rdma_k.start()
        rdma_v.start()

        qs = []
        for b in range(B):
            xb = x_ref[b].astype(jnp.bfloat16)
            wq = wq_ref[...].astype(jnp.bfloat16)
            qs.append(jnp.dot(xb, wq, preferred_element_type=jnp.float32))

        rdma_k.wait()
        rdma_v.wait()

        wo = wo_ref[...].astype(jnp.bfloat16)
        for b in range(B):
            rows = []
            for t in range(2):
                heads = []
                for h in range(HQ):
                    q = qs[b][t * BLK:(t + 1) * BLK, h * DH:(h + 1) * DH]
                    k_loc = k_ref[b, t * BLK:(t + 1) * BLK, h, :]
                    k_rem = ck_ref[b, t * BLK:(t + 1) * BLK, h, :]
                    kk = jnp.concatenate([k_loc, k_rem], axis=0)
                    s = lax.dot_general(
                        q.astype(jnp.bfloat16), kk.astype(jnp.bfloat16),
                        dimension_numbers=(((1,), (1,)), ((), ())),
                        preferred_element_type=jnp.float32,
                    ) * 0.125
                    s_max = jnp.max(s, axis=-1, keepdims=True)
                    w = jnp.exp(s - s_max)
                    w = w / jnp.sum(w, axis=-1, keepdims=True)
                    v_loc = v_ref[b, t * BLK:(t + 1) * BLK, h, :]
                    v_rem = cv_ref[b, t * BLK:(t + 1) * BLK, h, :]
                    vv = jnp.concatenate([v_loc, v_rem], axis=0)
                    ctx = jnp.dot(
                        w.astype(jnp.bfloat16), vv.astype(jnp.bfloat16),
                        preferred_element_type=jnp.float32,
                    )
                    heads.append(ctx)
                rows.append(jnp.concatenate(heads, axis=1))
            ctx_b = jnp.concatenate(rows, axis=0)
            out_ref[b] = jnp.dot(
                ctx_b.astype(jnp.bfloat16), wo,
                preferred_element_type=jnp.float32,
            )

    return pl.pallas_call(
        body,
        out_shape=jax.ShapeDtypeStruct((B, SQ_PER, D_MODEL), jnp.float32),
        in_specs=[pl.BlockSpec(memory_space=pltpu.VMEM)] * 5,
        out_specs=pl.BlockSpec(memory_space=pltpu.VMEM),
        scratch_shapes=[
            pltpu.VMEM((B, SKV_PER, HQ, DH), jnp.float32),
            pltpu.VMEM((B, SKV_PER, HQ, DH), jnp.float32),
            pltpu.SemaphoreType.DMA((2,)),
            pltpu.SemaphoreType.DMA((2,)),
        ],
        compiler_params=pltpu.CompilerParams(collective_id=0),
    )(x, Wq, K_ext, V_ext, Wo)


# device time: 13020 ns/iter; 1.8743x vs baseline; 1.8743x over previous
import jax
import jax.numpy as jnp
from jax import lax
from jax.experimental import pallas as pl
from jax.experimental.pallas import tpu as pltpu

N_DEV = 4
B, SQ_PER, SKV_PER, HQ, DH = 2, 128, 128, 4, 64
D_MODEL = 512
D_QK = HQ * DH
BLK = 64


def kernel(x, Wq, K_ext, V_ext, Wo):
    def body(x_ref, wq_ref, k_ref, v_ref, wo_ref, out_ref,
             ks_ref, vs_ref, ck_ref, cv_ref, send_sems, recv_sems):
        my = lax.axis_index("i")
        partner = (my + 2) % N_DEV

        for b in range(B):
            ks_ref[b] = k_ref[b].astype(jnp.bfloat16).reshape(SKV_PER, D_QK)
            vs_ref[b] = v_ref[b].astype(jnp.bfloat16).reshape(SKV_PER, D_QK)

        barrier_sem = pltpu.get_barrier_semaphore()
        pl.semaphore_signal(
            barrier_sem, inc=1,
            device_id=(partner,), device_id_type=pl.DeviceIdType.MESH,
        )
        pl.semaphore_wait(barrier_sem, 1)

        rdma_k = pltpu.make_async_remote_copy(
            src_ref=ks_ref, dst_ref=ck_ref,
            send_sem=send_sems.at[0], recv_sem=recv_sems.at[0],
            device_id=(partner,), device_id_type=pl.DeviceIdType.MESH,
        )
        rdma_v = pltpu.make_async_remote_copy(
            src_ref=vs_ref, dst_ref=cv_ref,
            send_sem=send_sems.at[1], recv_sem=recv_sems.at[1],
            device_id=(partner,), device_id_type=pl.DeviceIdType.MESH,
        )
        rdma_k.start()
        rdma_v.start()

        wq = wq_ref[...].astype(jnp.bfloat16)
        xx = x_ref[...].astype(jnp.bfloat16).reshape(B * SQ_PER, D_MODEL)
        q_all = jnp.dot(xx, wq, preferred_element_type=jnp.float32)
        q_all = q_all.reshape(B, SQ_PER, D_QK)

        ri = lax.broadcasted_iota(jnp.int32, (SQ_PER, 2 * SKV_PER), 0) // BLK
        ci = lax.broadcasted_iota(jnp.int32, (SQ_PER, 2 * SKV_PER), 1) // BLK
        mask = (ci % 2) == ri

        rdma_k.wait()
        rdma_v.wait()

        wo = wo_ref[...].astype(jnp.bfloat16)
        for b in range(B):
            heads = []
            for h in range(HQ):
                cols = pl.ds(h * DH, DH)
                q = q_all[b, :, h * DH:(h + 1) * DH].astype(jnp.bfloat16)
                kk = jnp.concatenate(
                    [ks_ref[b, :, cols], ck_ref[b, :, cols]], axis=0)
                s = lax.dot_general(
                    q, kk,
                    dimension_numbers=(((1,), (1,)), ((), ())),
                    preferred_element_type=jnp.float32,
                ) * 0.125
                s = jnp.where(mask, s, -1e9)
                s_max = jnp.max(s, axis=-1, keepdims=True)
                w = jnp.exp(s - s_max)
                w = w / jnp.sum(w, axis=-1, keepdims=True)
                vv = jnp.concatenate(
                    [vs_ref[b, :, cols], cv_ref[b, :, cols]], axis=0)
                ctx = jnp.dot(
                    w.astype(jnp.bfloat16), vv,
                    preferred_element_type=jnp.float32,
                )
                heads.append(ctx)
            ctx_b = jnp.concatenate(heads, axis=1)
            out_ref[b] = jnp.dot(
                ctx_b.astype(jnp.bfloat16), wo,
                preferred_element_type=jnp.float32,
            )

    return pl.pallas_call(
        body,
        out_shape=jax.ShapeDtypeStruct((B, SQ_PER, D_MODEL), jnp.float32),
        in_specs=[pl.BlockSpec(memory_space=pltpu.VMEM)] * 5,
        out_specs=pl.BlockSpec(memory_space=pltpu.VMEM),
        scratch_shapes=[
            pltpu.VMEM((B, SKV_PER, D_QK), jnp.bfloat16),
            pltpu.VMEM((B, SKV_PER, D_QK), jnp.bfloat16),
            pltpu.VMEM((B, SKV_PER, D_QK), jnp.bfloat16),
            pltpu.VMEM((B, SKV_PER, D_QK), jnp.bfloat16),
            pltpu.SemaphoreType.DMA((2,)),
            pltpu.SemaphoreType.DMA((2,)),
        ],
        compiler_params=pltpu.CompilerParams(collective_id=0),
    )(x, Wq, K_ext, V_ext, Wo)


# device time: 12026 ns/iter; 2.0292x vs baseline; 1.0827x over previous
import jax
import jax.numpy as jnp
from jax import lax
from jax.experimental import pallas as pl
from jax.experimental.pallas import tpu as pltpu

N_DEV = 4
B, SQ_PER, SKV_PER, HQ, DH = 2, 128, 128, 4, 64
D_MODEL = 512
D_QK = HQ * DH
BLK = 64


def kernel(x, Wq, K_ext, V_ext, Wo):
    def body(x_ref, wq_ref, k_ref, v_ref, wo_ref, out_ref,
             kv_ref, send_sem, recv_sem):
        my = lax.axis_index("i")
        partner = (my + 2) % N_DEV

        for b in range(B):
            kv_ref[0, b, :, :D_QK] = (
                k_ref[b].astype(jnp.bfloat16).reshape(SKV_PER, D_QK))
            kv_ref[0, b, :, D_QK:] = (
                v_ref[b].astype(jnp.bfloat16).reshape(SKV_PER, D_QK))

        barrier_sem = pltpu.get_barrier_semaphore()
        pl.semaphore_signal(
            barrier_sem, inc=1,
            device_id=(partner,), device_id_type=pl.DeviceIdType.MESH,
        )
        pl.semaphore_wait(barrier_sem, 1)

        rdma = pltpu.make_async_remote_copy(
            src_ref=kv_ref.at[0], dst_ref=kv_ref.at[1],
            send_sem=send_sem, recv_sem=recv_sem,
            device_id=(partner,), device_id_type=pl.DeviceIdType.MESH,
        )
        rdma.start()

        wq = wq_ref[...].astype(jnp.bfloat16)
        xx = x_ref[...].astype(jnp.bfloat16).reshape(B * SQ_PER, D_MODEL)
        q2 = (jnp.dot(xx, wq, preferred_element_type=jnp.float32)
              * 0.125).astype(jnp.bfloat16)

        ri = lax.broadcasted_iota(jnp.int32, (SQ_PER, 2 * SKV_PER), 0) // BLK
        ci = lax.broadcasted_iota(jnp.int32, (SQ_PER, 2 * SKV_PER), 1) // BLK
        mask = (ci % 2) == ri

        rdma.wait()

        ctx_rows = []
        for b in range(B):
            heads = []
            for h in range(HQ):
                q = q2[b * SQ_PER:(b + 1) * SQ_PER, h * DH:(h + 1) * DH]
                kk = kv_ref[:, b, :, h * DH:(h + 1) * DH].reshape(
                    2 * SKV_PER, DH)
                vv = kv_ref[:, b, :, D_QK + h * DH:D_QK + (h + 1) * DH
                            ].reshape(2 * SKV_PER, DH)
                s = lax.dot_general(
                    q, kk,
                    dimension_numbers=(((1,), (1,)), ((), ())),
                    preferred_element_type=jnp.float32,
                )
                w = jnp.exp(jnp.where(mask, s, -1e9))
                wsum = jnp.sum(w, axis=-1, keepdims=True)
                ctx = jnp.dot(
                    w.astype(jnp.bfloat16), vv,
                    preferred_element_type=jnp.float32,
                )
                heads.append((ctx * (1.0 / wsum)).astype(jnp.bfloat16))
            ctx_rows.append(jnp.concatenate(heads, axis=1))
        ctx_all = jnp.concatenate(ctx_rows, axis=0)

        wo = wo_ref[...].astype(jnp.bfloat16)
        out = jnp.dot(ctx_all, wo, preferred_element_type=jnp.float32)
        out_ref[...] = out.reshape(B, SQ_PER, D_MODEL)

    return pl.pallas_call(
        body,
        out_shape=jax.ShapeDtypeStruct((B, SQ_PER, D_MODEL), jnp.float32),
        in_specs=[pl.BlockSpec(memory_space=pltpu.VMEM)] * 5,
        out_specs=pl.BlockSpec(memory_space=pltpu.VMEM),
        scratch_shapes=[
            pltpu.VMEM((2, B, SKV_PER, 2 * D_QK), jnp.bfloat16),
            pltpu.SemaphoreType.DMA,
            pltpu.SemaphoreType.DMA,
        ],
        compiler_params=pltpu.CompilerParams(collective_id=0),
    )(x, Wq, K_ext, V_ext, Wo)


# device time: 11735 ns/iter; 2.0795x vs baseline; 1.0248x over previous
import jax
import jax.numpy as jnp
from jax import lax
from jax.experimental import pallas as pl
from jax.experimental.pallas import tpu as pltpu

N_DEV = 4
B, SQ_PER, SKV_PER, HQ, DH = 2, 128, 128, 4, 64
D_MODEL = 512
D_QK = HQ * DH
BLK = 64


def kernel(x, Wq, K_ext, V_ext, Wo):
    def body(x_ref, wq_ref, k_ref, v_ref, wo_ref, out_ref,
             kv_ref, send_sems, recv_sems):
        my = lax.axis_index("i")
        partner = (my + 2) % N_DEV

        barrier_sem = pltpu.get_barrier_semaphore()
        pl.semaphore_signal(
            barrier_sem, inc=1,
            device_id=(partner,), device_id_type=pl.DeviceIdType.MESH,
        )

        def send(b):
            r = pltpu.make_async_remote_copy(
                src_ref=kv_ref.at[0, b], dst_ref=kv_ref.at[1, b],
                send_sem=send_sems.at[b], recv_sem=recv_sems.at[b],
                device_id=(partner,), device_id_type=pl.DeviceIdType.MESH,
            )
            r.start()
            return r

        rdmas = []
        for b in range(B):
            kv_ref[0, b, :, :D_QK] = (
                k_ref[b].astype(jnp.bfloat16).reshape(SKV_PER, D_QK))
            kv_ref[0, b, :, D_QK:] = (
                v_ref[b].astype(jnp.bfloat16).reshape(SKV_PER, D_QK))
            if b == 0:
                pl.semaphore_wait(barrier_sem, 1)
            rdmas.append(send(b))

        wq = wq_ref[...].astype(jnp.bfloat16)
        xx = x_ref[...].astype(jnp.bfloat16).reshape(B * SQ_PER, D_MODEL)
        q2 = (jnp.dot(xx, wq, preferred_element_type=jnp.float32)
              * 0.125).astype(jnp.bfloat16)

        ri = lax.broadcasted_iota(jnp.int32, (SQ_PER, 2 * SKV_PER), 0) // BLK
        ci = lax.broadcasted_iota(jnp.int32, (SQ_PER, 2 * SKV_PER), 1) // BLK
        mask = (ci % 2) == ri

        ctx_rows = []
        for b in range(B):
            rdmas[b].wait_recv()
            heads = []
            for h in range(HQ):
                q = q2[b * SQ_PER:(b + 1) * SQ_PER, h * DH:(h + 1) * DH]
                kk = kv_ref[:, b, :, h * DH:(h + 1) * DH].reshape(
                    2 * SKV_PER, DH)
                vv = kv_ref[:, b, :, D_QK + h * DH:D_QK + (h + 1) * DH
                            ].reshape(2 * SKV_PER, DH)
                s = lax.dot_general(
                    q, kk,
                    dimension_numbers=(((1,), (1,)), ((), ())),
                    preferred_element_type=jnp.float32,
                )
                w = jnp.exp(jnp.where(mask, s, -1e9))
                wsum = jnp.sum(w, axis=-1, keepdims=True)
                ctx = jnp.dot(
                    w.astype(jnp.bfloat16), vv,
                    preferred_element_type=jnp.float32,
                )
                heads.append((ctx * (1.0 / wsum)).astype(jnp.bfloat16))
            ctx_rows.append(jnp.concatenate(heads, axis=1))
        ctx_all = jnp.concatenate(ctx_rows, axis=0)

        wo = wo_ref[...].astype(jnp.bfloat16)
        out = jnp.dot(ctx_all, wo, preferred_element_type=jnp.float32)
        out_ref[...] = out.reshape(B, SQ_PER, D_MODEL)

        for b in range(B):
            rdmas[b].wait_send()

    return pl.pallas_call(
        body,
        out_shape=jax.ShapeDtypeStruct((B, SQ_PER, D_MODEL), jnp.float32),
        in_specs=[pl.BlockSpec(memory_space=pltpu.VMEM)] * 5,
        out_specs=pl.BlockSpec(memory_space=pltpu.VMEM),
        scratch_shapes=[
            pltpu.VMEM((2, B, SKV_PER, 2 * D_QK), jnp.bfloat16),
            pltpu.SemaphoreType.DMA((B,)),
            pltpu.SemaphoreType.DMA((B,)),
        ],
        compiler_params=pltpu.CompilerParams(collective_id=0),
    )(x, Wq, K_ext, V_ext, Wo)
